# baseline (speedup 1.0000x reference)
import functools

import jax
import jax.numpy as jnp
from jax import lax
from jax.experimental import pallas as pl
from jax.experimental.pallas import tpu as pltpu

N = 10000
D = 128
BLOCK = 2000
GRID = N // BLOCK

_EPS = jax.random.normal(jax.random.PRNGKey(0), (64,), jnp.float32).reshape(1, 64)


def _block_kernel(nodes_ref, w1_ref, b1_ref, w2_ref, b2_ref,
                  wd1_ref, bd1_ref, wd2_ref, bd2_ref,
                  assign_ref, partial_ref):
    x = nodes_ref[...]
    h1 = jnp.maximum(
        jnp.dot(x, w1_ref[...], preferred_element_type=jnp.float32)
        + b1_ref[...], 0.0)
    h = jnp.dot(h1, w2_ref[...], preferred_element_type=jnp.float32) \
        + b2_ref[...]
    a = jnp.maximum(
        jnp.dot(h, wd1_ref[...], preferred_element_type=jnp.float32)
        + bd1_ref[...], 0.0)
    logits = jnp.dot(a, wd2_ref[...], preferred_element_type=jnp.float32) \
        + bd2_ref[...]
    m = jnp.max(logits, axis=-1, keepdims=True)
    e = jnp.exp(logits - m)
    assign = e / jnp.sum(e, axis=-1, keepdims=True)
    assign_ref[...] = assign
    partial_ref[0] = lax.dot_general(assign, h, (((0,), (0,)), ((), ())),
                                     preferred_element_type=jnp.float32)


def _head_kernel(partial_ref, wmu_ref, bmu_ref, wlv_ref, blv_ref,
                 wp1_ref, bp1_ref, wp2_ref, bp2_ref, eps_ref,
                 coarse_ref, mu_ref, lv_ref, pred_ref):
    coarse = jnp.sum(partial_ref[...], axis=0)
    coarse_ref[...] = coarse
    macro = jnp.mean(coarse, axis=0, keepdims=True)
    mu = jnp.dot(macro, wmu_ref[...],
                 preferred_element_type=jnp.float32) + bmu_ref[...]
    lv = jnp.dot(macro, wlv_ref[...],
                 preferred_element_type=jnp.float32) + blv_ref[...]
    std = jnp.exp(0.5 * lv)
    z = mu + eps_ref[...] * std
    p = jnp.maximum(
        jnp.dot(z, wp1_ref[...], preferred_element_type=jnp.float32)
        + bp1_ref[...], 0.0)
    pred = jnp.dot(p, wp2_ref[...],
                   preferred_element_type=jnp.float32) + bp2_ref[...]
    mu_ref[...] = mu
    lv_ref[...] = lv
    pred_ref[...] = pred


@functools.partial(jax.jit, static_argnames=("interpret",))
def _run(nodes, W_enc1, b_enc1, W_enc2, b_enc2,
         W_dec1, b_dec1, W_dec2, b_dec2,
         W_mu, b_mu, W_lv, b_lv,
         W_p1, b_p1, W_p2, b_p2, eps, interpret=False):
    full = lambda *shape: pl.BlockSpec(shape, lambda i: (0,) * len(shape))
    assignments, partials = pl.pallas_call(
        _block_kernel,
        grid=(GRID,),
        in_specs=[
            pl.BlockSpec((BLOCK, D), lambda i: (i, 0)),
            full(128, 128), full(1, 128),
            full(128, 128), full(1, 128),
            full(128, 32), full(1, 32),
            full(32, 64), full(1, 64),
        ],
        out_specs=[
            pl.BlockSpec((BLOCK, 64), lambda i: (i, 0)),
            pl.BlockSpec((1, 64, 128), lambda i: (i, 0, 0)),
        ],
        out_shape=[
            jax.ShapeDtypeStruct((N, 64), jnp.float32),
            jax.ShapeDtypeStruct((GRID, 64, 128), jnp.float32),
        ],
        compiler_params=pltpu.CompilerParams(
            dimension_semantics=("parallel",)),
        interpret=interpret,
    )(nodes, W_enc1, b_enc1.reshape(1, -1), W_enc2, b_enc2.reshape(1, -1),
      W_dec1, b_dec1.reshape(1, -1), W_dec2, b_dec2.reshape(1, -1))

    fullh = lambda *shape: pl.BlockSpec(shape, lambda: (0,) * len(shape))
    coarse_nodes, mu, lv, pred = pl.pallas_call(
        _head_kernel,
        in_specs=[
            fullh(GRID, 64, 128),
            fullh(128, 64), fullh(1, 64),
            fullh(128, 64), fullh(1, 64),
            fullh(64, 32), fullh(1, 32),
            fullh(32, 1), fullh(1, 1),
            fullh(1, 64),
        ],
        out_specs=[
            fullh(64, 128), fullh(1, 64), fullh(1, 64), fullh(1, 1),
        ],
        out_shape=[
            jax.ShapeDtypeStruct((64, 128), jnp.float32),
            jax.ShapeDtypeStruct((1, 64), jnp.float32),
            jax.ShapeDtypeStruct((1, 64), jnp.float32),
            jax.ShapeDtypeStruct((1, 1), jnp.float32),
        ],
        interpret=interpret,
    )(partials, W_mu, b_mu.reshape(1, -1), W_lv, b_lv.reshape(1, -1),
      W_p1, b_p1.reshape(1, -1), W_p2, b_p2.reshape(1, -1), eps)

    return (mu.reshape(-1), lv.reshape(-1), pred.reshape(-1),
            assignments, coarse_nodes)


def kernel(nodes, edges, senders, receivers,
           W_enc1, b_enc1, W_enc2, b_enc2,
           W_dec1, b_dec1, W_dec2, b_dec2,
           W_mu, b_mu, W_lv, b_lv,
           W_p1, b_p1, W_p2, b_p2):
    return _run(nodes, W_enc1, b_enc1, W_enc2, b_enc2,
                W_dec1, b_dec1, W_dec2, b_dec2,
                W_mu, b_mu, W_lv, b_lv,
                W_p1, b_p1, W_p2, b_p2, _EPS)


# speedup vs baseline: 1.8834x; 1.8834x over previous
import functools

import jax
import jax.numpy as jnp
from jax import lax
from jax.experimental import pallas as pl

N = 10000
D = 128
BLOCK = 2048
GRID = -(-N // BLOCK)


def _fused_kernel(nodes_ref, w1_ref, b1_ref, w2_ref, b2_ref,
                  wd1t_ref, bd1_ref, wd2_ref, bd2_ref,
                  wmut_ref, bmu_ref, wlvt_ref, blv_ref,
                  wp1t_ref, bp1_ref, wp2t_ref, bp2_ref, eps_ref,
                  assign_t_ref, coarse_ref, mu_ref, lv_ref, pred_ref):
    i = pl.program_id(0)

    x = nodes_ref[...]
    h1 = jnp.maximum(
        jnp.dot(x, w1_ref[...], preferred_element_type=jnp.float32)
        + b1_ref[...], 0.0)
    h = jnp.dot(h1, w2_ref[...], preferred_element_type=jnp.float32) \
        + b2_ref[...]
    a = jnp.maximum(
        lax.dot_general(h, wd1t_ref[...], (((1,), (1,)), ((), ())),
                        preferred_element_type=jnp.float32)
        + bd1_ref[...], 0.0)
    logits = jnp.dot(a, wd2_ref[...], preferred_element_type=jnp.float32) \
        + bd2_ref[...]
    m = jnp.max(logits, axis=-1, keepdims=True)
    e = jnp.exp(logits - m)
    assign = e / jnp.sum(e, axis=-1, keepdims=True)
    valid = (i * BLOCK + lax.broadcasted_iota(jnp.int32, (BLOCK, 1), 0)) < N
    assign = jnp.where(valid, assign, 0.0)
    hm = jnp.where(valid, h, 0.0)
    assign_t = jnp.swapaxes(assign, 0, 1)
    assign_t_ref[...] = assign_t

    partial = lax.dot_general(assign_t, hm, (((1,), (0,)), ((), ())),
                              preferred_element_type=jnp.float32)

    @pl.when(i == 0)
    def _():
        coarse_ref[...] = partial

    @pl.when(i > 0)
    def _():
        coarse_ref[...] += partial

    @pl.when(i == GRID - 1)
    def _():
        macro = jnp.mean(coarse_ref[...], axis=0, keepdims=True)
        mu = lax.dot_general(macro, wmut_ref[...], (((1,), (1,)), ((), ())),
                             preferred_element_type=jnp.float32) + bmu_ref[...]
        lv = lax.dot_general(macro, wlvt_ref[...], (((1,), (1,)), ((), ())),
                             preferred_element_type=jnp.float32) + blv_ref[...]
        std = jnp.exp(0.5 * lv)
        z = mu + eps_ref[...] * std
        p = jnp.maximum(
            lax.dot_general(z, wp1t_ref[...], (((1,), (1,)), ((), ())),
                            preferred_element_type=jnp.float32)
            + bp1_ref[...], 0.0)
        pred = jnp.sum(p * wp2t_ref[...], axis=1, keepdims=True) \
            + bp2_ref[...]
        mu_ref[...] = mu
        lv_ref[...] = lv
        pred_ref[...] = pred


@functools.partial(jax.jit, static_argnames=("interpret",))
def _run(nodes, W_enc1, b_enc1, W_enc2, b_enc2,
         W_dec1, b_dec1, W_dec2, b_dec2,
         W_mu, b_mu, W_lv, b_lv,
         W_p1, b_p1, W_p2, b_p2, interpret=False):
    eps = jax.random.normal(jax.random.PRNGKey(0), (1, 64), jnp.float32)
    full = lambda *shape: pl.BlockSpec(shape, lambda i: (0,) * len(shape))
    out = pl.pallas_call(
        _fused_kernel,
        grid=(GRID,),
        in_specs=[
            pl.BlockSpec((BLOCK, D), lambda i: (i, 0)),
            full(128, 128), full(1, 128),
            full(128, 128), full(1, 128),
            full(32, 128), full(1, 32),
            full(32, 64), full(1, 64),
            full(64, 128), full(1, 64),
            full(64, 128), full(1, 64),
            full(32, 64), full(1, 32),
            full(1, 32), full(1, 1),
            full(1, 64),
        ],
        out_specs=[
            pl.BlockSpec((64, BLOCK), lambda i: (0, i)),
            full(64, 128),
            full(1, 64), full(1, 64), full(1, 1),
        ],
        out_shape=[
            jax.ShapeDtypeStruct((64, N), jnp.float32),
            jax.ShapeDtypeStruct((64, 128), jnp.float32),
            jax.ShapeDtypeStruct((1, 64), jnp.float32),
            jax.ShapeDtypeStruct((1, 64), jnp.float32),
            jax.ShapeDtypeStruct((1, 1), jnp.float32),
        ],
        interpret=interpret,
    )(nodes, W_enc1, b_enc1.reshape(1, -1), W_enc2, b_enc2.reshape(1, -1),
      W_dec1.T, b_dec1.reshape(1, -1), W_dec2, b_dec2.reshape(1, -1),
      W_mu.T, b_mu.reshape(1, -1), W_lv.T, b_lv.reshape(1, -1),
      W_p1.T, b_p1.reshape(1, -1), W_p2.T, b_p2.reshape(1, -1), eps)
    assign_t, coarse_nodes, mu, lv, pred = out
    return (mu.reshape(-1), lv.reshape(-1), pred.reshape(-1),
            assign_t.T, coarse_nodes)


def kernel(nodes, edges, senders, receivers,
           W_enc1, b_enc1, W_enc2, b_enc2,
           W_dec1, b_dec1, W_dec2, b_dec2,
           W_mu, b_mu, W_lv, b_lv,
           W_p1, b_p1, W_p2, b_p2):
    return _run(nodes, W_enc1, b_enc1, W_enc2, b_enc2,
                W_dec1, b_dec1, W_dec2, b_dec2,
                W_mu, b_mu, W_lv, b_lv,
                W_p1, b_p1, W_p2, b_p2)


# speedup vs baseline: 2.0042x; 1.0641x over previous
import functools

import jax
import jax.numpy as jnp
import numpy as np
from jax import lax
from jax.experimental import pallas as pl

N = 10000
D = 128
BLOCK = 2048
GRID = -(-N // BLOCK)

try:
    _EPS = np.asarray(
        jax.random.normal(jax.random.PRNGKey(0), (64,), jnp.float32)
    ).reshape(1, 64)
except Exception:
    _EPS = None


def _fused_kernel(nodes_ref, w1_ref, b1_ref, w2_ref, b2_ref,
                  wd1t_ref, bd1_ref, wd2t_ref, bd2t_ref,
                  wmut_ref, bmu_ref, wlvt_ref, blv_ref,
                  wp1t_ref, bp1_ref, wp2t_ref, bp2_ref, eps_ref,
                  assign_t_ref, coarse_ref, mu_ref, lv_ref, pred_ref):
    i = pl.program_id(0)

    x = nodes_ref[...]
    h1 = jnp.maximum(
        jnp.dot(x, w1_ref[...], preferred_element_type=jnp.float32)
        + b1_ref[...], 0.0)
    h = jnp.dot(h1, w2_ref[...], preferred_element_type=jnp.float32) \
        + b2_ref[...]
    a = jnp.maximum(
        lax.dot_general(h, wd1t_ref[...], (((1,), (1,)), ((), ())),
                        preferred_element_type=jnp.float32)
        + bd1_ref[...], 0.0)
    logits_t = lax.dot_general(wd2t_ref[...], a, (((1,), (1,)), ((), ())),
                               preferred_element_type=jnp.float32) \
        + bd2t_ref[...]
    m = jnp.max(logits_t, axis=0, keepdims=True)
    e = jnp.exp(logits_t - m)
    assign_t = e / jnp.sum(e, axis=0, keepdims=True)
    base = i * BLOCK
    valid_c = (base + lax.broadcasted_iota(jnp.int32, (1, BLOCK), 1)) < N
    valid_r = (base + lax.broadcasted_iota(jnp.int32, (BLOCK, 1), 0)) < N
    assign_t = jnp.where(valid_c, assign_t, 0.0)
    hm = jnp.where(valid_r, h, 0.0)
    assign_t_ref[...] = assign_t

    partial = lax.dot_general(assign_t, hm, (((1,), (0,)), ((), ())),
                              preferred_element_type=jnp.float32)

    @pl.when(i == 0)
    def _():
        coarse_ref[...] = partial

    @pl.when(i > 0)
    def _():
        coarse_ref[...] += partial

    @pl.when(i == GRID - 1)
    def _():
        macro = jnp.mean(coarse_ref[...], axis=0, keepdims=True)
        mu = lax.dot_general(macro, wmut_ref[...], (((1,), (1,)), ((), ())),
                             preferred_element_type=jnp.float32) + bmu_ref[...]
        lv = lax.dot_general(macro, wlvt_ref[...], (((1,), (1,)), ((), ())),
                             preferred_element_type=jnp.float32) + blv_ref[...]
        std = jnp.exp(0.5 * lv)
        z = mu + eps_ref[...] * std
        p = jnp.maximum(
            lax.dot_general(z, wp1t_ref[...], (((1,), (1,)), ((), ())),
                            preferred_element_type=jnp.float32)
            + bp1_ref[...], 0.0)
        pred = jnp.sum(p * wp2t_ref[...], axis=1, keepdims=True) \
            + bp2_ref[...]
        mu_ref[...] = mu
        lv_ref[...] = lv
        pred_ref[...] = pred


@functools.partial(jax.jit, static_argnames=("interpret",))
def _run(nodes, W_enc1, b_enc1, W_enc2, b_enc2,
         W_dec1, b_dec1, W_dec2, b_dec2,
         W_mu, b_mu, W_lv, b_lv,
         W_p1, b_p1, W_p2, b_p2, interpret=False):
    if _EPS is None:
        eps = jax.random.normal(jax.random.PRNGKey(0), (64,),
                                jnp.float32).reshape(1, 64)
    else:
        eps = _EPS
    full = lambda *shape: pl.BlockSpec(shape, lambda i: (0,) * len(shape))
    out = pl.pallas_call(
        _fused_kernel,
        grid=(GRID,),
        in_specs=[
            pl.BlockSpec((BLOCK, D), lambda i: (i, 0)),
            full(128, 128), full(1, 128),
            full(128, 128), full(1, 128),
            full(32, 128), full(1, 32),
            full(64, 32), full(64, 1),
            full(64, 128), full(1, 64),
            full(64, 128), full(1, 64),
            full(32, 64), full(1, 32),
            full(1, 32), full(1, 1),
            full(1, 64),
        ],
        out_specs=[
            pl.BlockSpec((64, BLOCK), lambda i: (0, i)),
            full(64, 128),
            full(1, 64), full(1, 64), full(1, 1),
        ],
        out_shape=[
            jax.ShapeDtypeStruct((64, N), jnp.float32),
            jax.ShapeDtypeStruct((64, 128), jnp.float32),
            jax.ShapeDtypeStruct((1, 64), jnp.float32),
            jax.ShapeDtypeStruct((1, 64), jnp.float32),
            jax.ShapeDtypeStruct((1, 1), jnp.float32),
        ],
        interpret=interpret,
    )(nodes, W_enc1, b_enc1.reshape(1, -1), W_enc2, b_enc2.reshape(1, -1),
      W_dec1.T, b_dec1.reshape(1, -1), W_dec2.T, b_dec2.reshape(-1, 1),
      W_mu.T, b_mu.reshape(1, -1), W_lv.T, b_lv.reshape(1, -1),
      W_p1.T, b_p1.reshape(1, -1), W_p2.T, b_p2.reshape(1, -1), eps)
    assign_t, coarse_nodes, mu, lv, pred = out
    return (mu.reshape(-1), lv.reshape(-1), pred.reshape(-1),
            assign_t.T, coarse_nodes)


def kernel(nodes, edges, senders, receivers,
           W_enc1, b_enc1, W_enc2, b_enc2,
           W_dec1, b_dec1, W_dec2, b_dec2,
           W_mu, b_mu, W_lv, b_lv,
           W_p1, b_p1, W_p2, b_p2):
    return _run(nodes, W_enc1, b_enc1, W_enc2, b_enc2,
                W_dec1, b_dec1, W_dec2, b_dec2,
                W_mu, b_mu, W_lv, b_lv,
                W_p1, b_p1, W_p2, b_p2)


# speedup vs baseline: 2.5518x; 1.2733x over previous
import functools

import jax
import jax.numpy as jnp
import numpy as np
from jax import lax
from jax.experimental import pallas as pl

N = 10000
D = 128
BLOCK = 2048
GRID = -(-N // BLOCK)

try:
    _EPS = np.asarray(
        jax.random.normal(jax.random.PRNGKey(0), (64,), jnp.float32)
    ).reshape(1, 64)
except Exception:
    _EPS = None


def _fused_kernel(nodes_ref, w1_ref, b1_ref, w2_ref, b2_ref,
                  wd1t_ref, bd1_ref, wd2_ref, bd2_ref,
                  wmut_ref, bmu_ref, wlvt_ref, blv_ref,
                  wp1t_ref, bp1_ref, wp2t_ref, bp2_ref, eps_ref,
                  assign_t_ref, coarse_ref, mu_ref, lv_ref, pred_ref):
    i = pl.program_id(0)

    x = nodes_ref[...]
    h1 = jnp.maximum(
        jnp.dot(x, w1_ref[...], preferred_element_type=jnp.float32)
        + b1_ref[...], 0.0)
    h = jnp.dot(h1, w2_ref[...], preferred_element_type=jnp.float32) \
        + b2_ref[...]
    a = jnp.maximum(
        lax.dot_general(h, wd1t_ref[...], (((1,), (1,)), ((), ())),
                        preferred_element_type=jnp.float32)
        + bd1_ref[...], 0.0)
    wd2t = jnp.swapaxes(wd2_ref[...], 0, 1)
    bd2t = jnp.swapaxes(bd2_ref[...], 0, 1)
    logits_t = lax.dot_general(wd2t, a, (((1,), (1,)), ((), ())),
                               preferred_element_type=jnp.float32) \
        + bd2t
    m = jnp.max(logits_t, axis=0, keepdims=True)
    e = jnp.exp(logits_t - m)
    assign_t = e / jnp.sum(e, axis=0, keepdims=True)
    base = i * BLOCK
    valid_c = (base + lax.broadcasted_iota(jnp.int32, (1, BLOCK), 1)) < N
    valid_r = (base + lax.broadcasted_iota(jnp.int32, (BLOCK, 1), 0)) < N
    assign_t = jnp.where(valid_c, assign_t, 0.0)
    hm = jnp.where(valid_r, h, 0.0)
    assign_t_ref[...] = assign_t

    partial = lax.dot_general(assign_t, hm, (((1,), (0,)), ((), ())),
                              preferred_element_type=jnp.float32)

    @pl.when(i == 0)
    def _():
        coarse_ref[...] = partial

    @pl.when(i > 0)
    def _():
        coarse_ref[...] += partial

    @pl.when(i == GRID - 1)
    def _():
        macro = jnp.mean(coarse_ref[...], axis=0, keepdims=True)
        mu = lax.dot_general(macro, wmut_ref[...], (((1,), (1,)), ((), ())),
                             preferred_element_type=jnp.float32) + bmu_ref[...]
        lv = lax.dot_general(macro, wlvt_ref[...], (((1,), (1,)), ((), ())),
                             preferred_element_type=jnp.float32) + blv_ref[...]
        std = jnp.exp(0.5 * lv)
        z = mu + eps_ref[...] * std
        p = jnp.maximum(
            lax.dot_general(z, wp1t_ref[...], (((1,), (1,)), ((), ())),
                            preferred_element_type=jnp.float32)
            + bp1_ref[...], 0.0)
        pred = jnp.sum(p * wp2t_ref[...], axis=1, keepdims=True) \
            + bp2_ref[...]
        mu_ref[...] = mu
        lv_ref[...] = lv
        pred_ref[...] = pred


@functools.partial(jax.jit, static_argnames=("interpret",))
def _run(nodes, W_enc1, b_enc1, W_enc2, b_enc2,
         W_dec1, b_dec1, W_dec2, b_dec2,
         W_mu, b_mu, W_lv, b_lv,
         W_p1, b_p1, W_p2, b_p2, interpret=False):
    if _EPS is None:
        eps = jax.random.normal(jax.random.PRNGKey(0), (64,),
                                jnp.float32).reshape(1, 64)
    else:
        eps = _EPS
    full = lambda *shape: pl.BlockSpec(shape, lambda i: (0,) * len(shape))
    out = pl.pallas_call(
        _fused_kernel,
        grid=(GRID,),
        in_specs=[
            pl.BlockSpec((BLOCK, D), lambda i: (i, 0)),
            full(128, 128), full(1, 128),
            full(128, 128), full(1, 128),
            full(32, 128), full(1, 32),
            full(32, 64), full(1, 64),
            full(64, 128), full(1, 64),
            full(64, 128), full(1, 64),
            full(32, 64), full(1, 32),
            full(1, 32), full(1, 1),
            full(1, 64),
        ],
        out_specs=[
            pl.BlockSpec((64, BLOCK), lambda i: (0, i)),
            full(64, 128),
            full(1, 64), full(1, 64), full(1, 1),
        ],
        out_shape=[
            jax.ShapeDtypeStruct((64, N), jnp.float32),
            jax.ShapeDtypeStruct((64, 128), jnp.float32),
            jax.ShapeDtypeStruct((1, 64), jnp.float32),
            jax.ShapeDtypeStruct((1, 64), jnp.float32),
            jax.ShapeDtypeStruct((1, 1), jnp.float32),
        ],
        interpret=interpret,
    )(nodes, W_enc1, b_enc1.reshape(1, -1), W_enc2, b_enc2.reshape(1, -1),
      W_dec1.T, b_dec1.reshape(1, -1), W_dec2, b_dec2.reshape(1, -1),
      W_mu.T, b_mu.reshape(1, -1), W_lv.T, b_lv.reshape(1, -1),
      W_p1.T, b_p1.reshape(1, -1), W_p2.T, b_p2.reshape(1, -1), eps)
    assign_t, coarse_nodes, mu, lv, pred = out
    return (mu.reshape(-1), lv.reshape(-1), pred.reshape(-1),
            assign_t.T, coarse_nodes)


def kernel(nodes, edges, senders, receivers,
           W_enc1, b_enc1, W_enc2, b_enc2,
           W_dec1, b_dec1, W_dec2, b_dec2,
           W_mu, b_mu, W_lv, b_lv,
           W_p1, b_p1, W_p2, b_p2):
    return _run(nodes, W_enc1, b_enc1, W_enc2, b_enc2,
                W_dec1, b_dec1, W_dec2, b_dec2,
                W_mu, b_mu, W_lv, b_lv,
                W_p1, b_p1, W_p2, b_p2)


# speedup vs baseline: 2.6146x; 1.0246x over previous
import functools

import jax
import jax.numpy as jnp
import numpy as np
from jax import lax
from jax.experimental import pallas as pl

N = 10000
D = 128
BLOCK = 1792
PAIRS = 3

try:
    _EPS = np.asarray(
        jax.random.normal(jax.random.PRNGKey(0), (64,), jnp.float32)
    ).reshape(1, 64)
except Exception:
    _EPS = None


def _half(x, w1, b1, w2, b2, wd1t, bd1, wd2t, bd2t, base):
    h1 = jnp.maximum(
        jnp.dot(x, w1, preferred_element_type=jnp.float32) + b1, 0.0)
    h = jnp.dot(h1, w2, preferred_element_type=jnp.float32) + b2
    a = jnp.maximum(
        lax.dot_general(h, wd1t, (((1,), (1,)), ((), ())),
                        preferred_element_type=jnp.float32) + bd1, 0.0)
    logits_t = lax.dot_general(wd2t, a, (((1,), (1,)), ((), ())),
                               preferred_element_type=jnp.float32) + bd2t
    m = jnp.max(logits_t, axis=0, keepdims=True)
    e = jnp.exp(logits_t - m)
    assign_t = e / jnp.sum(e, axis=0, keepdims=True)
    valid_c = (base + lax.broadcasted_iota(jnp.int32, (1, BLOCK), 1)) < N
    valid_r = (base + lax.broadcasted_iota(jnp.int32, (BLOCK, 1), 0)) < N
    assign_t = jnp.where(valid_c, assign_t, 0.0)
    hm = jnp.where(valid_r, h, 0.0)
    partial = lax.dot_general(assign_t, hm, (((1,), (0,)), ((), ())),
                              preferred_element_type=jnp.float32)
    return assign_t, partial


def _fused_kernel(nodes_a_ref, nodes_b_ref, w1_ref, b1_ref, w2_ref, b2_ref,
                  wd1t_ref, bd1_ref, wd2_ref, bd2_ref,
                  wmut_ref, bmu_ref, wlvt_ref, blv_ref,
                  wp1t_ref, bp1_ref, wp2t_ref, bp2_ref, eps_ref,
                  assign_t_ref, coarse_ref, mu_ref, lv_ref, pred_ref):
    i = pl.program_id(0)

    wd2t = jnp.swapaxes(wd2_ref[...], 0, 1)
    bd2t = jnp.swapaxes(bd2_ref[...], 0, 1)
    args = (w1_ref[...], b1_ref[...], w2_ref[...], b2_ref[...],
            wd1t_ref[...], bd1_ref[...], wd2t, bd2t)
    at_a, part_a = _half(nodes_a_ref[...], *args, (2 * i) * BLOCK)
    at_b, part_b = _half(nodes_b_ref[...], *args, (2 * i + 1) * BLOCK)
    assign_t_ref[:, :BLOCK] = at_a
    assign_t_ref[:, BLOCK:] = at_b
    partial = part_a + part_b

    @pl.when(i == 0)
    def _():
        coarse_ref[...] = partial

    @pl.when(i > 0)
    def _():
        coarse_ref[...] += partial

    @pl.when(i == PAIRS - 1)
    def _():
        macro = jnp.mean(coarse_ref[...], axis=0, keepdims=True)
        mu = lax.dot_general(macro, wmut_ref[...], (((1,), (1,)), ((), ())),
                             preferred_element_type=jnp.float32) + bmu_ref[...]
        lv = lax.dot_general(macro, wlvt_ref[...], (((1,), (1,)), ((), ())),
                             preferred_element_type=jnp.float32) + blv_ref[...]
        std = jnp.exp(0.5 * lv)
        z = mu + eps_ref[...] * std
        p = jnp.maximum(
            lax.dot_general(z, wp1t_ref[...], (((1,), (1,)), ((), ())),
                            preferred_element_type=jnp.float32)
            + bp1_ref[...], 0.0)
        pred = jnp.sum(p * wp2t_ref[...], axis=1, keepdims=True) \
            + bp2_ref[...]
        mu_ref[...] = mu
        lv_ref[...] = lv
        pred_ref[...] = pred


@functools.partial(jax.jit, static_argnames=("interpret",))
def _run(nodes, W_enc1, b_enc1, W_enc2, b_enc2,
         W_dec1, b_dec1, W_dec2, b_dec2,
         W_mu, b_mu, W_lv, b_lv,
         W_p1, b_p1, W_p2, b_p2, interpret=False):
    if _EPS is None:
        eps = jax.random.normal(jax.random.PRNGKey(0), (64,),
                                jnp.float32).reshape(1, 64)
    else:
        eps = _EPS
    full = lambda *shape: pl.BlockSpec(shape, lambda i: (0,) * len(shape))
    out = pl.pallas_call(
        _fused_kernel,
        grid=(PAIRS,),
        in_specs=[
            pl.BlockSpec((BLOCK, D), lambda i: (2 * i, 0)),
            pl.BlockSpec((BLOCK, D), lambda i: (2 * i + 1, 0)),
            full(128, 128), full(1, 128),
            full(128, 128), full(1, 128),
            full(32, 128), full(1, 32),
            full(32, 64), full(1, 64),
            full(64, 128), full(1, 64),
            full(64, 128), full(1, 64),
            full(32, 64), full(1, 32),
            full(1, 32), full(1, 1),
            full(1, 64),
        ],
        out_specs=[
            pl.BlockSpec((64, 2 * BLOCK), lambda i: (0, i)),
            full(64, 128),
            full(1, 64), full(1, 64), full(1, 1),
        ],
        out_shape=[
            jax.ShapeDtypeStruct((64, N), jnp.float32),
            jax.ShapeDtypeStruct((64, 128), jnp.float32),
            jax.ShapeDtypeStruct((1, 64), jnp.float32),
            jax.ShapeDtypeStruct((1, 64), jnp.float32),
            jax.ShapeDtypeStruct((1, 1), jnp.float32),
        ],
        interpret=interpret,
    )(nodes, nodes, W_enc1, b_enc1.reshape(1, -1), W_enc2, b_enc2.reshape(1, -1),
      W_dec1.T, b_dec1.reshape(1, -1), W_dec2, b_dec2.reshape(1, -1),
      W_mu.T, b_mu.reshape(1, -1), W_lv.T, b_lv.reshape(1, -1),
      W_p1.T, b_p1.reshape(1, -1), W_p2.T, b_p2.reshape(1, -1), eps)
    assign_t, coarse_nodes, mu, lv, pred = out
    return (mu.reshape(-1), lv.reshape(-1), pred.reshape(-1),
            assign_t.T, coarse_nodes)


def kernel(nodes, edges, senders, receivers,
           W_enc1, b_enc1, W_enc2, b_enc2,
           W_dec1, b_dec1, W_dec2, b_dec2,
           W_mu, b_mu, W_lv, b_lv,
           W_p1, b_p1, W_p2, b_p2):
    return _run(nodes, W_enc1, b_enc1, W_enc2, b_enc2,
                W_dec1, b_dec1, W_dec2, b_dec2,
                W_mu, b_mu, W_lv, b_lv,
                W_p1, b_p1, W_p2, b_p2)
